# baseline (speedup 1.0000x reference)
import functools

import jax
import jax.numpy as jnp
from jax import lax
from jax.experimental import pallas as pl
from jax.experimental.pallas import tpu as pltpu
from jax.experimental.pallas import tpu_sc as plsc

N = 10000
NP = 10240
D = 128
H = 256
OUT = 60
OUTP = 64
B = 4096
E = 320000
EP = 327680
ER = EP // 128
DUMMY = 10208
NC = 2
NS = 16
SLC = NP // NS
K = 8
KP = 4
TOTAL = OUT + OUT + 1024 + 1024 + 512 + 512 + 954

F32 = jnp.float32


def _mesh():
  return plsc.VectorSubcoreMesh(core_axis_name="c", subcore_axis_name="s")


def _sc_deg(dstp):
  rows_per_tile = ER // (NC * NS)
  nsg = rows_per_tile // K

  @functools.partial(
      pl.kernel,
      mesh=_mesh(),
      compiler_params=pltpu.CompilerParams(use_tc_tiling_on_sc=False),
      out_type=jax.ShapeDtypeStruct((NC * NP,), F32),
      scratch_types=[
          pltpu.VMEM((K, 128), jnp.int32),
          pltpu.VMEM((128,), F32),
          pltpu.VMEM((SLC,), F32),
          pltpu.VMEM_SHARED((NP,), F32),
          pltpu.SemaphoreType.DMA,
      ],
  )
  def deg_kernel(dstp_hbm, out_hbm, idx_v, ones_v, zbuf_v, deg_sh, sem):
    c = lax.axis_index("c")
    s = lax.axis_index("s")

    @pl.loop(0, SLC // 16)
    def _(i):
      zbuf_v[pl.ds(i * 16, 16)] = jnp.zeros((16,), F32)

    @pl.loop(0, 128 // 16)
    def _(i):
      ones_v[pl.ds(i * 16, 16)] = jnp.ones((16,), F32)

    pltpu.sync_copy(zbuf_v, deg_sh.at[pl.ds(s * SLC, SLC)])
    plsc.subcore_barrier()

    base = (c * NS + s) * rows_per_tile

    @pl.loop(0, nsg)
    def _(g):
      pltpu.sync_copy(dstp_hbm.at[pl.ds(base + g * K, K)], idx_v)
      for j in range(K):
        pltpu.make_async_copy(ones_v, deg_sh.at[idx_v.at[j]], sem).start(
            add=True)
      for j in range(K):
        pltpu.make_async_copy(ones_v, deg_sh.at[idx_v.at[j]], sem).wait()

    plsc.subcore_barrier()
    pltpu.sync_copy(deg_sh.at[pl.ds(s * SLC, SLC)],
                    out_hbm.at[pl.ds(c * NP + s * SLC, SLC)])

  return deg_kernel(dstp)


def _tc_prep(x_pad, degp):
  def body(deg_ref, x_ref, dinv_ref, xs_ref):
    deg = deg_ref[0, :] + deg_ref[1, :]
    dinv = lax.rsqrt(jnp.maximum(deg, 1.0))
    dinv_ref[...] = dinv[:, None]
    xs = x_ref[...] * dinv[:, None]
    xs_ref[...] = jnp.stack([xs[:, :64], xs[:, 64:]], axis=0)

  return pl.pallas_call(
      body,
      grid=(NS,),
      in_specs=[
          pl.BlockSpec((NC, SLC), lambda i: (0, i)),
          pl.BlockSpec((SLC, D), lambda i: (i, 0)),
      ],
      out_specs=[
          pl.BlockSpec((SLC, 1), lambda i: (i, 0)),
          pl.BlockSpec((NC, SLC, 64), lambda i: (0, i, 0)),
      ],
      out_shape=[
          jax.ShapeDtypeStruct((NP, 1), F32),
          jax.ShapeDtypeStruct((NC, NP, 64), F32),
      ],
  )(degp, x_pad)


def _zero_slice(zbuf, agg_sh, s, w):
  @pl.loop(0, 128)
  def _(i):
    for k2 in range(w // 16):
      zbuf[i, pl.ds(k2 * 16, 16)] = jnp.zeros((16,), F32)

  for t in range(SLC // 128):
    pltpu.sync_copy(zbuf, agg_sh.at[pl.ds(s * SLC + t * 128, 128)])


def _edge_pipeline(c, s, src_hbm, dst_hbm, tab_hbm, isrc, idst, rows0,
                   rows1, agg_sh, sem_g, sem_s0, sem_s1):
  base = s * (ER // NS)
  nsg = (ER // NS) // KP
  bufs = ((rows0, sem_s0), (rows1, sem_s1))

  def do_sg(sg, p, first):
    rows_b, sem_sb = bufs[p]
    if not first:
      for j in range(KP):
        pltpu.make_async_copy(rows_b.at[j], agg_sh.at[idst.at[p, j]],
                              sem_sb).wait()
    r0 = base + sg * KP
    pltpu.sync_copy(src_hbm.at[c, pl.ds(r0, KP)], isrc.at[p])
    pltpu.sync_copy(dst_hbm.at[pl.ds(r0, KP)], idst.at[p])
    for j in range(KP):
      pltpu.make_async_copy(tab_hbm.at[isrc.at[p, j]], rows_b.at[j],
                            sem_g).start()
    for j in range(KP):
      pltpu.make_async_copy(tab_hbm.at[isrc.at[p, j]], rows_b.at[j],
                            sem_g).wait()
      pltpu.make_async_copy(rows_b.at[j], agg_sh.at[idst.at[p, j]],
                            sem_sb).start(add=True)

  do_sg(0, 0, True)
  do_sg(1, 1, True)

  @pl.loop(0, nsg // 2 - 1)
  def _(t):
    do_sg(2 * t + 2, 0, False)
    do_sg(2 * t + 3, 1, False)

  for p in range(2):
    rows_b, sem_sb = bufs[p]
    for j in range(KP):
      pltpu.make_async_copy(rows_b.at[j], agg_sh.at[idst.at[p, j]],
                            sem_sb).wait()


def _sc_agg1(srcp2, dstp, xs_flat):
  W = 64

  @functools.partial(
      pl.kernel,
      mesh=_mesh(),
      compiler_params=pltpu.CompilerParams(use_tc_tiling_on_sc=False),
      out_type=jax.ShapeDtypeStruct((NC, NP, 64), F32),
      scratch_types=[
          pltpu.VMEM((2, KP, 128), jnp.int32),
          pltpu.VMEM((2, KP, 128), jnp.int32),
          pltpu.VMEM((KP, 128, W), F32),
          pltpu.VMEM((KP, 128, W), F32),
          pltpu.VMEM((128, W), F32),
          pltpu.VMEM_SHARED((NP, W), F32),
          pltpu.SemaphoreType.DMA,
          pltpu.SemaphoreType.DMA,
          pltpu.SemaphoreType.DMA,
      ],
  )
  def k(srcp2_hbm, dstp_hbm, xs_hbm, out_hbm, isrc, idst, rows0, rows1,
        zbuf, agg_sh, sem_g, sem_s0, sem_s1):
    c = lax.axis_index("c")
    s = lax.axis_index("s")

    _zero_slice(zbuf, agg_sh, s, W)
    plsc.subcore_barrier()
    _edge_pipeline(c, s, srcp2_hbm, dstp_hbm, xs_hbm, isrc, idst, rows0,
                   rows1, agg_sh, sem_g, sem_s0, sem_s1)
    plsc.subcore_barrier()
    pltpu.sync_copy(agg_sh.at[pl.ds(s * SLC, SLC)],
                    out_hbm.at[c, pl.ds(s * SLC, SLC)])

  return k(srcp2, dstp, xs_flat)


def _tc_mm(agg1, dinv, w1, w2p):
  def body(agg_ref, dinv_ref, w1_ref, w2_ref, gs_ref):
    dv = dinv_ref[...]
    av = agg_ref[...]
    a = jnp.concatenate([av[0], av[1]], axis=1) * dv
    h = jnp.dot(a, w1_ref[...], precision=lax.Precision.HIGHEST,
                preferred_element_type=F32)
    h = jnp.maximum(h, 0.0)
    g = jnp.dot(h, w2_ref[...], precision=lax.Precision.HIGHEST,
                preferred_element_type=F32)
    gs = g * dv
    gs_ref[...] = jnp.stack([gs[:, :32], gs[:, 32:]], axis=0)

  return pl.pallas_call(
      body,
      grid=(NS,),
      in_specs=[
          pl.BlockSpec((NC, SLC, 64), lambda i: (0, i, 0)),
          pl.BlockSpec((SLC, 1), lambda i: (i, 0)),
          pl.BlockSpec((D, H), lambda i: (0, 0)),
          pl.BlockSpec((H, OUTP), lambda i: (0, 0)),
      ],
      out_specs=pl.BlockSpec((NC, SLC, 32), lambda i: (0, i, 0)),
      out_shape=jax.ShapeDtypeStruct((NC, NP, 32), F32),
  )(agg1, dinv, w1, w2p)


def _sc_agg2(srcp2, dstp, gs_flat, dinv, d1r, d2r):
  W = 32
  prows_per_tile = B // 128 // NS

  @functools.partial(
      pl.kernel,
      mesh=_mesh(),
      compiler_params=pltpu.CompilerParams(use_tc_tiling_on_sc=False),
      out_type=jax.ShapeDtypeStruct((2, NC, B, 32), F32),
      scratch_types=[
          pltpu.VMEM((2, KP, 128), jnp.int32),
          pltpu.VMEM((2, KP, 128), jnp.int32),
          pltpu.VMEM((KP, 128, W), F32),
          pltpu.VMEM((KP, 128, W), F32),
          pltpu.VMEM((128, W), F32),
          pltpu.VMEM((SLC, W), F32),
          pltpu.VMEM((SLC,), F32),
          pltpu.VMEM((1, 128), jnp.int32),
          pltpu.VMEM((128, W), F32),
          pltpu.VMEM_SHARED((NP, W), F32),
          pltpu.SemaphoreType.DMA,
          pltpu.SemaphoreType.DMA,
          pltpu.SemaphoreType.DMA,
      ],
  )
  def k(srcp2_hbm, dstp_hbm, gs_hbm, dinv_hbm, d1r_hbm, d2r_hbm, pf_hbm,
        isrc, idst, rows0, rows1, zbuf, sbuf, dinv_v, pidx, prow_v,
        agg_sh, sem_g, sem_s0, sem_s1):
    c = lax.axis_index("c")
    s = lax.axis_index("s")

    _zero_slice(zbuf, agg_sh, s, W)
    plsc.subcore_barrier()
    _edge_pipeline(c, s, srcp2_hbm, dstp_hbm, gs_hbm, isrc, idst, rows0,
                   rows1, agg_sh, sem_g, sem_s0, sem_s1)
    plsc.subcore_barrier()

    pltpu.sync_copy(agg_sh.at[pl.ds(s * SLC, SLC)], sbuf)
    pltpu.sync_copy(dinv_hbm.at[pl.ds(s * SLC, SLC)], dinv_v)

    @pl.loop(0, SLC // 16)
    def _(q):
      dv16 = dinv_v[pl.ds(q * 16, 16)]
      for j in range(16):
        r = q * 16 + j
        dv = dv16[j]
        for k2 in range(W // 16):
          sbuf[r, pl.ds(k2 * 16, 16)] = sbuf[r, pl.ds(k2 * 16, 16)] * dv

    pltpu.sync_copy(sbuf, agg_sh.at[pl.ds(s * SLC, SLC)])
    plsc.subcore_barrier()

    for g in range(prows_per_tile):
      pr = s * prows_per_tile + g
      for d, dref in enumerate((d1r_hbm, d2r_hbm)):
        pltpu.sync_copy(dref.at[pr], pidx)
        pltpu.make_async_copy(agg_sh.at[pidx.at[0]], prow_v, sem_g).start()
        pltpu.make_async_copy(agg_sh.at[pidx.at[0]], prow_v, sem_g).wait()
        pltpu.sync_copy(prow_v, pf_hbm.at[d, c, pl.ds(pr * 128, 128)])

  return k(srcp2, dstp, gs_flat, dinv, d1r, d2r)


def _tc_assemble(pf, fp1, fp2, dti1, dti2, cell):
  RB = 64

  def body(pf_ref, fp1_ref, fp2_ref, dti1_ref, dti2_ref, cell_ref, out_ref):
    v = pf_ref[...]
    d1 = jnp.concatenate([v[0, 0], v[0, 1]], axis=1)
    d2 = jnp.concatenate([v[1, 0], v[1, 1]], axis=1)
    out_ref[:, 0:60] = d1[:, 0:60]
    out_ref[:, 60:120] = d2[:, 0:60]
    out_ref[:, 120:1144] = fp1_ref[...]
    out_ref[:, 1144:2168] = fp2_ref[...]
    out_ref[:, 2168:2680] = dti1_ref[...]
    out_ref[:, 2680:3192] = dti2_ref[...]
    out_ref[:, 3192:4146] = cell_ref[...]

  return pl.pallas_call(
      body,
      grid=(B // RB,),
      in_specs=[
          pl.BlockSpec((2, NC, RB, 32), lambda i: (0, 0, i, 0)),
          pl.BlockSpec((RB, 1024), lambda i: (i, 0)),
          pl.BlockSpec((RB, 1024), lambda i: (i, 0)),
          pl.BlockSpec((RB, 512), lambda i: (i, 0)),
          pl.BlockSpec((RB, 512), lambda i: (i, 0)),
          pl.BlockSpec((RB, 954), lambda i: (i, 0)),
      ],
      out_specs=pl.BlockSpec((RB, TOTAL), lambda i: (i, 0)),
      out_shape=jax.ShapeDtypeStruct((B, TOTAL), F32),
  )(pf, fp1, fp2, dti1, dti2, cell)


def kernel(drug1_idx, drug2_idx, drug1_fp, drug2_fp, drug1_dti, drug2_dti,
           cell_feat, x, edge_index, W1, W2):
  src = edge_index[0].astype(jnp.int32)
  dst = edge_index[1].astype(jnp.int32)
  pad = jnp.full((EP - E,), DUMMY, jnp.int32)
  srcp = jnp.concatenate([src, pad]).reshape(ER, 128)
  dstp = jnp.concatenate([dst, pad]).reshape(ER, 128)
  srcp2 = jnp.stack([srcp, srcp + NP])
  x_pad = jnp.pad(x, ((0, NP - N), (0, 0)))
  w2p = jnp.pad(W2, ((0, 0), (0, OUTP - OUT)))
  d1r = jnp.ravel(drug1_idx).astype(jnp.int32).reshape(B // 128, 1, 128)
  d2r = jnp.ravel(drug2_idx).astype(jnp.int32).reshape(B // 128, 1, 128)

  degp = _sc_deg(dstp).reshape(NC, NP)
  dinv, xs = _tc_prep(x_pad, degp)
  xs_flat = xs.reshape(2 * NP, 64)
  agg1 = _sc_agg1(srcp2, dstp, xs_flat)
  gs = _tc_mm(agg1, dinv, W1, w2p)
  gs_flat = gs.reshape(2 * NP, 32)
  pf = _sc_agg2(srcp2, dstp, gs_flat, dinv.reshape(NP), d1r, d2r)
  return _tc_assemble(pf, drug1_fp, drug2_fp, drug1_dti, drug2_dti, cell_feat)


# speedup vs baseline: 1.6063x; 1.6063x over previous
import functools

import jax
import jax.numpy as jnp
from jax import lax
from jax.experimental import pallas as pl
from jax.experimental.pallas import tpu as pltpu
from jax.experimental.pallas import tpu_sc as plsc

N = 10000
NP = 10240
D = 128
H = 256
OUT = 60
OUTP = 64
B = 4096
E = 320000
EP = 327680
ER = EP // 128
DUMMY = 10208
NC = 2
NS = 16
SLC = NP // NS
K = 8
KP = 4
TOTAL = OUT + OUT + 1024 + 1024 + 512 + 512 + 954

F32 = jnp.float32


def _mesh():
  return plsc.VectorSubcoreMesh(core_axis_name="c", subcore_axis_name="s")


def _sc_deg(dstp):
  rows_per_tile = ER // (NC * NS)
  nsg = rows_per_tile // K

  @functools.partial(
      pl.kernel,
      mesh=_mesh(),
      compiler_params=pltpu.CompilerParams(use_tc_tiling_on_sc=False),
      out_type=jax.ShapeDtypeStruct((NC * NP,), F32),
      scratch_types=[
          pltpu.VMEM((K, 128), jnp.int32),
          pltpu.VMEM((128,), F32),
          pltpu.VMEM((SLC,), F32),
          pltpu.VMEM_SHARED((NP,), F32),
          pltpu.SemaphoreType.DMA,
      ],
  )
  def deg_kernel(dstp_hbm, out_hbm, idx_v, ones_v, zbuf_v, deg_sh, sem):
    c = lax.axis_index("c")
    s = lax.axis_index("s")

    @pl.loop(0, SLC // 16)
    def _(i):
      zbuf_v[pl.ds(i * 16, 16)] = jnp.zeros((16,), F32)

    @pl.loop(0, 128 // 16)
    def _(i):
      ones_v[pl.ds(i * 16, 16)] = jnp.ones((16,), F32)

    pltpu.sync_copy(zbuf_v, deg_sh.at[pl.ds(s * SLC, SLC)])
    plsc.subcore_barrier()

    base = (c * NS + s) * rows_per_tile

    @pl.loop(0, nsg)
    def _(g):
      pltpu.sync_copy(dstp_hbm.at[pl.ds(base + g * K, K)], idx_v)
      for j in range(K):
        pltpu.make_async_copy(ones_v, deg_sh.at[idx_v.at[j]], sem).start(
            add=True)
      for j in range(K):
        pltpu.make_async_copy(ones_v, deg_sh.at[idx_v.at[j]], sem).wait()

    plsc.subcore_barrier()
    pltpu.sync_copy(deg_sh.at[pl.ds(s * SLC, SLC)],
                    out_hbm.at[pl.ds(c * NP + s * SLC, SLC)])

  return deg_kernel(dstp)


def _tc_prep(x_pad, degp):
  def body(deg_ref, x_ref, dinv_ref, xs_ref):
    deg = deg_ref[0, :] + deg_ref[1, :]
    dinv = lax.rsqrt(jnp.maximum(deg, 1.0))
    dinv_ref[...] = dinv[:, None]
    xs = x_ref[...] * dinv[:, None]
    xs_ref[...] = jnp.stack([xs[:, :64], xs[:, 64:]], axis=0)

  return pl.pallas_call(
      body,
      grid=(NS,),
      in_specs=[
          pl.BlockSpec((NC, SLC), lambda i: (0, i)),
          pl.BlockSpec((SLC, D), lambda i: (i, 0)),
      ],
      out_specs=[
          pl.BlockSpec((SLC, 1), lambda i: (i, 0)),
          pl.BlockSpec((NC, SLC, 64), lambda i: (0, i, 0)),
      ],
      out_shape=[
          jax.ShapeDtypeStruct((NP, 1), F32),
          jax.ShapeDtypeStruct((NC, NP, 64), F32),
      ],
  )(degp, x_pad)


def _zero_slice(zbuf, agg_sh, s, w):
  @pl.loop(0, 128)
  def _(i):
    for k2 in range(w // 16):
      zbuf[i, pl.ds(k2 * 16, 16)] = jnp.zeros((16,), F32)

  for t in range(SLC // 128):
    pltpu.sync_copy(zbuf, agg_sh.at[pl.ds(s * SLC + t * 128, 128)])


def _edge_pipeline(c, s, src_hbm, dst_hbm, tab_sh, isrc, idst, rows0,
                   rows1, zbuf, agg_sh, sem_g, sem_s0, sem_s1):
  base = s * (ER // NS)
  IG = 8
  nsg = (ER // NS) // IG
  bufs = ((rows0, sem_s0), (rows1, sem_s1))

  for r in range(2):
    for kk in range(8):
      idst[0, r, pl.ds(kk * 16, 16)] = jnp.zeros((16,), jnp.int32)
  for p in range(2):
    _, sem_sb = bufs[p]
    for j in range(2):
      pltpu.make_async_copy(zbuf, agg_sh.at[idst.at[0, j]],
                            sem_sb).start(add=True)

  def do_sg(g, pg):
    r0 = base + g * IG
    pltpu.sync_copy(src_hbm.at[pl.ds(r0, IG)], isrc)
    pltpu.sync_copy(dst_hbm.at[pl.ds(r0, IG)], idst.at[pg])
    for w in range(4):
      rows_b, sem_sb = bufs[w % 2]
      for j in range(2):
        pltpu.make_async_copy(rows_b.at[j], agg_sh.at[idst.at[0, j]],
                              sem_sb).wait()
      for j in range(2):
        pltpu.make_async_copy(tab_sh.at[isrc.at[2 * w + j]], rows_b.at[j],
                              sem_g).start()
      for j in range(2):
        pltpu.make_async_copy(tab_sh.at[isrc.at[2 * w + j]], rows_b.at[j],
                              sem_g).wait()
        pltpu.make_async_copy(rows_b.at[j], agg_sh.at[idst.at[pg, 2 * w + j]],
                              sem_sb).start(add=True)

  @pl.loop(0, nsg // 2)
  def _(t):
    do_sg(2 * t, 0)
    do_sg(2 * t + 1, 1)

  for p in range(2):
    rows_b, sem_sb = bufs[p]
    for j in range(2):
      pltpu.make_async_copy(rows_b.at[j], agg_sh.at[idst.at[0, j]],
                            sem_sb).wait()


def _sc_agg1(srcp, dstp, xs_flat):
  W = 64

  @functools.partial(
      pl.kernel,
      mesh=_mesh(),
      compiler_params=pltpu.CompilerParams(use_tc_tiling_on_sc=False),
      out_type=jax.ShapeDtypeStruct((NC, NP, 64), F32),
      scratch_types=[
          pltpu.VMEM((8, 128), jnp.int32),
          pltpu.VMEM((2, 8, 128), jnp.int32),
          pltpu.VMEM((2, 128, W), F32),
          pltpu.VMEM((2, 128, W), F32),
          pltpu.VMEM((128, W), F32),
          pltpu.VMEM_SHARED((NP, W), F32),
          pltpu.VMEM_SHARED((NP, W), F32),
          pltpu.SemaphoreType.DMA,
          pltpu.SemaphoreType.DMA,
          pltpu.SemaphoreType.DMA,
      ],
  )
  def k(srcp_hbm, dstp_hbm, xs_hbm, out_hbm, isrc, idst, rows0, rows1,
        zbuf, tab_sh, agg_sh, sem_g, sem_s0, sem_s1):
    c = lax.axis_index("c")
    s = lax.axis_index("s")

    _zero_slice(zbuf, agg_sh, s, W)
    pltpu.sync_copy(xs_hbm.at[pl.ds(c * NP + s * SLC, SLC)],
                    tab_sh.at[pl.ds(s * SLC, SLC)])
    plsc.subcore_barrier()
    _edge_pipeline(c, s, srcp_hbm, dstp_hbm, tab_sh, isrc, idst, rows0,
                   rows1, zbuf, agg_sh, sem_g, sem_s0, sem_s1)
    plsc.subcore_barrier()
    pltpu.sync_copy(agg_sh.at[pl.ds(s * SLC, SLC)],
                    out_hbm.at[c, pl.ds(s * SLC, SLC)])

  return k(srcp, dstp, xs_flat)


def _tc_mm(agg1, dinv, w1, w2p):
  def body(agg_ref, dinv_ref, w1_ref, w2_ref, gs_ref):
    dv = dinv_ref[...]
    av = agg_ref[...]
    a = jnp.concatenate([av[0], av[1]], axis=1) * dv
    h = jnp.dot(a, w1_ref[...], precision=lax.Precision.HIGHEST,
                preferred_element_type=F32)
    h = jnp.maximum(h, 0.0)
    g = jnp.dot(h, w2_ref[...], precision=lax.Precision.HIGHEST,
                preferred_element_type=F32)
    gs = g * dv
    gs_ref[...] = jnp.stack([gs[:, :32], gs[:, 32:]], axis=0)

  return pl.pallas_call(
      body,
      grid=(NS,),
      in_specs=[
          pl.BlockSpec((NC, SLC, 64), lambda i: (0, i, 0)),
          pl.BlockSpec((SLC, 1), lambda i: (i, 0)),
          pl.BlockSpec((D, H), lambda i: (0, 0)),
          pl.BlockSpec((H, OUTP), lambda i: (0, 0)),
      ],
      out_specs=pl.BlockSpec((NC, SLC, 32), lambda i: (0, i, 0)),
      out_shape=jax.ShapeDtypeStruct((NC, NP, 32), F32),
  )(agg1, dinv, w1, w2p)


def _sc_agg2(srcp, dstp, gs_flat, dinv, d1r, d2r):
  W = 32
  prows_per_tile = B // 128 // NS

  @functools.partial(
      pl.kernel,
      mesh=_mesh(),
      compiler_params=pltpu.CompilerParams(use_tc_tiling_on_sc=False),
      out_type=jax.ShapeDtypeStruct((2, NC, B, 32), F32),
      scratch_types=[
          pltpu.VMEM((8, 128), jnp.int32),
          pltpu.VMEM((2, 8, 128), jnp.int32),
          pltpu.VMEM((2, 128, W), F32),
          pltpu.VMEM((2, 128, W), F32),
          pltpu.VMEM((128, W), F32),
          pltpu.VMEM((SLC, W), F32),
          pltpu.VMEM((SLC,), F32),
          pltpu.VMEM((1, 128), jnp.int32),
          pltpu.VMEM((128, W), F32),
          pltpu.VMEM_SHARED((NP, W), F32),
          pltpu.VMEM_SHARED((NP, W), F32),
          pltpu.SemaphoreType.DMA,
          pltpu.SemaphoreType.DMA,
          pltpu.SemaphoreType.DMA,
      ],
  )
  def k(srcp_hbm, dstp_hbm, gs_hbm, dinv_hbm, d1r_hbm, d2r_hbm, pf_hbm,
        isrc, idst, rows0, rows1, zbuf, sbuf, dinv_v, pidx, prow_v,
        tab_sh, agg_sh, sem_g, sem_s0, sem_s1):
    c = lax.axis_index("c")
    s = lax.axis_index("s")

    _zero_slice(zbuf, agg_sh, s, W)
    pltpu.sync_copy(gs_hbm.at[pl.ds(c * NP + s * SLC, SLC)],
                    tab_sh.at[pl.ds(s * SLC, SLC)])
    plsc.subcore_barrier()
    _edge_pipeline(c, s, srcp_hbm, dstp_hbm, tab_sh, isrc, idst, rows0,
                   rows1, zbuf, agg_sh, sem_g, sem_s0, sem_s1)
    plsc.subcore_barrier()

    pltpu.sync_copy(agg_sh.at[pl.ds(s * SLC, SLC)], sbuf)
    pltpu.sync_copy(dinv_hbm.at[pl.ds(s * SLC, SLC)], dinv_v)

    @pl.loop(0, SLC // 16)
    def _(q):
      dv16 = dinv_v[pl.ds(q * 16, 16)]
      for j in range(16):
        r = q * 16 + j
        dv = dv16[j]
        for k2 in range(W // 16):
          sbuf[r, pl.ds(k2 * 16, 16)] = sbuf[r, pl.ds(k2 * 16, 16)] * dv

    pltpu.sync_copy(sbuf, agg_sh.at[pl.ds(s * SLC, SLC)])
    plsc.subcore_barrier()

    for g in range(prows_per_tile):
      pr = s * prows_per_tile + g
      for d, dref in enumerate((d1r_hbm, d2r_hbm)):
        pltpu.sync_copy(dref.at[pr], pidx)
        pltpu.make_async_copy(agg_sh.at[pidx.at[0]], prow_v, sem_g).start()
        pltpu.make_async_copy(agg_sh.at[pidx.at[0]], prow_v, sem_g).wait()
        pltpu.sync_copy(prow_v, pf_hbm.at[d, c, pl.ds(pr * 128, 128)])

  return k(srcp, dstp, gs_flat, dinv, d1r, d2r)


def _tc_assemble(pf, fp1, fp2, dti1, dti2, cell):
  RB = 64

  def body(pf_ref, fp1_ref, fp2_ref, dti1_ref, dti2_ref, cell_ref, out_ref):
    v = pf_ref[...]
    d1 = jnp.concatenate([v[0, 0], v[0, 1]], axis=1)
    d2 = jnp.concatenate([v[1, 0], v[1, 1]], axis=1)
    out_ref[:, 0:60] = d1[:, 0:60]
    out_ref[:, 60:120] = d2[:, 0:60]
    out_ref[:, 120:1144] = fp1_ref[...]
    out_ref[:, 1144:2168] = fp2_ref[...]
    out_ref[:, 2168:2680] = dti1_ref[...]
    out_ref[:, 2680:3192] = dti2_ref[...]
    out_ref[:, 3192:4146] = cell_ref[...]

  return pl.pallas_call(
      body,
      grid=(B // RB,),
      in_specs=[
          pl.BlockSpec((2, NC, RB, 32), lambda i: (0, 0, i, 0)),
          pl.BlockSpec((RB, 1024), lambda i: (i, 0)),
          pl.BlockSpec((RB, 1024), lambda i: (i, 0)),
          pl.BlockSpec((RB, 512), lambda i: (i, 0)),
          pl.BlockSpec((RB, 512), lambda i: (i, 0)),
          pl.BlockSpec((RB, 954), lambda i: (i, 0)),
      ],
      out_specs=pl.BlockSpec((RB, TOTAL), lambda i: (i, 0)),
      out_shape=jax.ShapeDtypeStruct((B, TOTAL), F32),
  )(pf, fp1, fp2, dti1, dti2, cell)


def kernel(drug1_idx, drug2_idx, drug1_fp, drug2_fp, drug1_dti, drug2_dti,
           cell_feat, x, edge_index, W1, W2):
  src = edge_index[0].astype(jnp.int32)
  dst = edge_index[1].astype(jnp.int32)
  pad = jnp.full((EP - E,), DUMMY, jnp.int32)
  srcp = jnp.concatenate([src, pad]).reshape(ER, 128)
  dstp = jnp.concatenate([dst, pad]).reshape(ER, 128)
  x_pad = jnp.pad(x, ((0, NP - N), (0, 0)))
  w2p = jnp.pad(W2, ((0, 0), (0, OUTP - OUT)))
  d1r = jnp.ravel(drug1_idx).astype(jnp.int32).reshape(B // 128, 1, 128)
  d2r = jnp.ravel(drug2_idx).astype(jnp.int32).reshape(B // 128, 1, 128)

  degp = _sc_deg(dstp).reshape(NC, NP)
  dinv, xs = _tc_prep(x_pad, degp)
  xs_flat = xs.reshape(2 * NP, 64)
  agg1 = _sc_agg1(srcp, dstp, xs_flat)
  gs = _tc_mm(agg1, dinv, W1, w2p)
  gs_flat = gs.reshape(2 * NP, 32)
  pf = _sc_agg2(srcp, dstp, gs_flat, dinv.reshape(NP), d1r, d2r)
  return _tc_assemble(pf, drug1_fp, drug2_fp, drug1_dti, drug2_dti, cell_feat)


# speedup vs baseline: 1.7960x; 1.1181x over previous
import functools

import jax
import jax.numpy as jnp
from jax import lax
from jax.experimental import pallas as pl
from jax.experimental.pallas import tpu as pltpu
from jax.experimental.pallas import tpu_sc as plsc

N = 10000
NP = 10240
D = 128
H = 256
OUT = 60
OUTP = 64
B = 4096
E = 320000
EP = 327680
ER = EP // 128
DUMMY = 10208
NC = 2
NS = 16
SLC = NP // NS
K = 8
KP = 4
TOTAL = OUT + OUT + 1024 + 1024 + 512 + 512 + 954

F32 = jnp.float32


def _mesh():
  return plsc.VectorSubcoreMesh(core_axis_name="c", subcore_axis_name="s")


def _sc_deg(dstp):
  rows_per_tile = ER // (NC * NS)
  nsg = rows_per_tile // K

  @functools.partial(
      pl.kernel,
      mesh=_mesh(),
      compiler_params=pltpu.CompilerParams(use_tc_tiling_on_sc=False),
      out_type=jax.ShapeDtypeStruct((NC * NP,), F32),
      scratch_types=[
          pltpu.VMEM((K, 128), jnp.int32),
          pltpu.VMEM((128,), F32),
          pltpu.VMEM((SLC,), F32),
          pltpu.VMEM_SHARED((NP,), F32),
          pltpu.SemaphoreType.DMA,
      ],
  )
  def deg_kernel(dstp_hbm, out_hbm, idx_v, ones_v, zbuf_v, deg_sh, sem):
    c = lax.axis_index("c")
    s = lax.axis_index("s")

    @pl.loop(0, SLC // 16)
    def _(i):
      zbuf_v[pl.ds(i * 16, 16)] = jnp.zeros((16,), F32)

    @pl.loop(0, 128 // 16)
    def _(i):
      ones_v[pl.ds(i * 16, 16)] = jnp.ones((16,), F32)

    pltpu.sync_copy(zbuf_v, deg_sh.at[pl.ds(s * SLC, SLC)])
    plsc.subcore_barrier()

    base = (c * NS + s) * rows_per_tile

    @pl.loop(0, nsg)
    def _(g):
      pltpu.sync_copy(dstp_hbm.at[pl.ds(base + g * K, K)], idx_v)
      for j in range(K):
        pltpu.make_async_copy(ones_v, deg_sh.at[idx_v.at[j]], sem).start(
            add=True)
      for j in range(K):
        pltpu.make_async_copy(ones_v, deg_sh.at[idx_v.at[j]], sem).wait()

    plsc.subcore_barrier()
    pltpu.sync_copy(deg_sh.at[pl.ds(s * SLC, SLC)],
                    out_hbm.at[pl.ds(c * NP + s * SLC, SLC)])

  return deg_kernel(dstp)


def _tc_prep(x_pad, degp):
  def body(deg_ref, x_ref, dinv_ref, xs_ref):
    deg = deg_ref[0, :] + deg_ref[1, :]
    dinv = lax.rsqrt(jnp.maximum(deg, 1.0))
    dinv_ref[...] = dinv[:, None]
    xs = x_ref[...] * dinv[:, None]
    xs_ref[...] = jnp.stack([xs[:, :64], xs[:, 64:]], axis=0)

  return pl.pallas_call(
      body,
      grid=(NS,),
      in_specs=[
          pl.BlockSpec((NC, SLC), lambda i: (0, i)),
          pl.BlockSpec((SLC, D), lambda i: (i, 0)),
      ],
      out_specs=[
          pl.BlockSpec((SLC, 1), lambda i: (i, 0)),
          pl.BlockSpec((NC, SLC, 64), lambda i: (0, i, 0)),
      ],
      out_shape=[
          jax.ShapeDtypeStruct((NP, 1), F32),
          jax.ShapeDtypeStruct((NC, NP, 64), F32),
      ],
  )(degp, x_pad)


def _zero_slice(zbuf, agg_sh, s, w):
  @pl.loop(0, 128)
  def _(i):
    for k2 in range(w // 16):
      zbuf[i, pl.ds(k2 * 16, 16)] = jnp.zeros((16,), F32)

  for t in range(SLC // 128):
    pltpu.sync_copy(zbuf, agg_sh.at[pl.ds(s * SLC + t * 128, 128)])


def _edge_pipeline(c, s, src_hbm, dst_hbm, tab_sh, isrc, idst, rows0,
                   rows1, zbuf, agg_sh, sem_g, sem_s0, sem_s1):
  base = s * (ER // NS)
  IG = 8
  nsg = (ER // NS) // IG
  bufs = ((rows0, sem_s0), (rows1, sem_s1))

  for r in range(2):
    for kk in range(8):
      idst[0, r, pl.ds(kk * 16, 16)] = jnp.zeros((16,), jnp.int32)
  for p in range(2):
    _, sem_sb = bufs[p]
    for j in range(2):
      pltpu.make_async_copy(zbuf, agg_sh.at[idst.at[0, j]],
                            sem_sb).start(add=True)

  def do_sg(g, pg):
    r0 = base + g * IG
    pltpu.sync_copy(src_hbm.at[pl.ds(r0, IG)], isrc)
    pltpu.sync_copy(dst_hbm.at[pl.ds(r0, IG)], idst.at[pg])
    for w in range(4):
      rows_b, sem_sb = bufs[w % 2]
      for j in range(2):
        pltpu.make_async_copy(rows_b.at[j], agg_sh.at[idst.at[0, j]],
                              sem_sb).wait()
      for j in range(2):
        pltpu.make_async_copy(tab_sh.at[isrc.at[2 * w + j]], rows_b.at[j],
                              sem_g).start()
      for j in range(2):
        pltpu.make_async_copy(tab_sh.at[isrc.at[2 * w + j]], rows_b.at[j],
                              sem_g).wait()
        pltpu.make_async_copy(rows_b.at[j], agg_sh.at[idst.at[pg, 2 * w + j]],
                              sem_sb).start(add=True)

  @pl.loop(0, nsg // 2)
  def _(t):
    do_sg(2 * t, 0)
    do_sg(2 * t + 1, 1)

  for p in range(2):
    rows_b, sem_sb = bufs[p]
    for j in range(2):
      pltpu.make_async_copy(rows_b.at[j], agg_sh.at[idst.at[0, j]],
                            sem_sb).wait()


def _sc_agg1(srcp, dstp, xs_flat):
  W = 64

  @functools.partial(
      pl.kernel,
      mesh=_mesh(),
      compiler_params=pltpu.CompilerParams(use_tc_tiling_on_sc=False),
      out_type=jax.ShapeDtypeStruct((NC, NP, 64), F32),
      scratch_types=[
          pltpu.VMEM((8, 128), jnp.int32),
          pltpu.VMEM((2, 8, 128), jnp.int32),
          pltpu.VMEM((2, 128, W), F32),
          pltpu.VMEM((2, 128, W), F32),
          pltpu.VMEM((128, W), F32),
          pltpu.VMEM_SHARED((NP, W), F32),
          pltpu.VMEM_SHARED((NP, W), F32),
          pltpu.SemaphoreType.DMA,
          pltpu.SemaphoreType.DMA,
          pltpu.SemaphoreType.DMA,
      ],
  )
  def k(srcp_hbm, dstp_hbm, xs_hbm, out_hbm, isrc, idst, rows0, rows1,
        zbuf, tab_sh, agg_sh, sem_g, sem_s0, sem_s1):
    c = lax.axis_index("c")
    s = lax.axis_index("s")

    _zero_slice(zbuf, agg_sh, s, W)
    pltpu.sync_copy(xs_hbm.at[pl.ds(c * NP + s * SLC, SLC)],
                    tab_sh.at[pl.ds(s * SLC, SLC)])
    plsc.subcore_barrier()
    _edge_pipeline(c, s, srcp_hbm, dstp_hbm, tab_sh, isrc, idst, rows0,
                   rows1, zbuf, agg_sh, sem_g, sem_s0, sem_s1)
    plsc.subcore_barrier()
    pltpu.sync_copy(agg_sh.at[pl.ds(s * SLC, SLC)],
                    out_hbm.at[c, pl.ds(s * SLC, SLC)])

  return k(srcp, dstp, xs_flat)


def _tc_mm(agg1, dinv, w1, w2p):
  def body(agg_ref, dinv_ref, w1_ref, w2_ref, gs_ref):
    dv = dinv_ref[...]
    av = agg_ref[...]
    a = jnp.concatenate([av[0], av[1]], axis=1) * dv
    h = jnp.dot(a, w1_ref[...], precision=lax.Precision.HIGHEST,
                preferred_element_type=F32)
    h = jnp.maximum(h, 0.0)
    g = jnp.dot(h, w2_ref[...], precision=lax.Precision.HIGHEST,
                preferred_element_type=F32)
    gs = g * dv
    gs_ref[...] = jnp.stack([gs[:, :32], gs[:, 32:]], axis=0)

  return pl.pallas_call(
      body,
      grid=(NS,),
      in_specs=[
          pl.BlockSpec((NC, SLC, 64), lambda i: (0, i, 0)),
          pl.BlockSpec((SLC, 1), lambda i: (i, 0)),
          pl.BlockSpec((D, H), lambda i: (0, 0)),
          pl.BlockSpec((H, OUTP), lambda i: (0, 0)),
      ],
      out_specs=pl.BlockSpec((NC, SLC, 32), lambda i: (0, i, 0)),
      out_shape=jax.ShapeDtypeStruct((NC, NP, 32), F32),
  )(agg1, dinv, w1, w2p)


def _sc_agg2(srcp, dstp, gs_flat, dinv, d1r, d2r):
  W = 32
  prows_per_tile = B // 128 // NS

  @functools.partial(
      pl.kernel,
      mesh=_mesh(),
      compiler_params=pltpu.CompilerParams(use_tc_tiling_on_sc=False),
      out_type=jax.ShapeDtypeStruct((2, NC, B, 32), F32),
      scratch_types=[
          pltpu.VMEM((8, 128), jnp.int32),
          pltpu.VMEM((2, 8, 128), jnp.int32),
          pltpu.VMEM((2, 128, W), F32),
          pltpu.VMEM((2, 128, W), F32),
          pltpu.VMEM((128, W), F32),
          pltpu.VMEM((SLC, W), F32),
          pltpu.VMEM((SLC,), F32),
          pltpu.VMEM((1, 128), jnp.int32),
          pltpu.VMEM((128, W), F32),
          pltpu.VMEM_SHARED((NP, W), F32),
          pltpu.VMEM_SHARED((NP, W), F32),
          pltpu.SemaphoreType.DMA,
          pltpu.SemaphoreType.DMA,
          pltpu.SemaphoreType.DMA,
      ],
  )
  def k(srcp_hbm, dstp_hbm, gs_hbm, dinv_hbm, d1r_hbm, d2r_hbm, pf_hbm,
        isrc, idst, rows0, rows1, zbuf, sbuf, dinv_v, pidx, prow_v,
        tab_sh, agg_sh, sem_g, sem_s0, sem_s1):
    c = lax.axis_index("c")
    s = lax.axis_index("s")

    _zero_slice(zbuf, agg_sh, s, W)
    pltpu.sync_copy(gs_hbm.at[pl.ds(c * NP + s * SLC, SLC)],
                    tab_sh.at[pl.ds(s * SLC, SLC)])
    plsc.subcore_barrier()
    _edge_pipeline(c, s, srcp_hbm, dstp_hbm, tab_sh, isrc, idst, rows0,
                   rows1, zbuf, agg_sh, sem_g, sem_s0, sem_s1)
    plsc.subcore_barrier()

    pltpu.sync_copy(agg_sh.at[pl.ds(s * SLC, SLC)], sbuf)
    pltpu.sync_copy(dinv_hbm.at[pl.ds(s * SLC, SLC)], dinv_v)

    @pl.loop(0, SLC // 16)
    def _(q):
      dv16 = dinv_v[pl.ds(q * 16, 16)]
      for j in range(16):
        r = q * 16 + j
        dv = dv16[j]
        for k2 in range(W // 16):
          sbuf[r, pl.ds(k2 * 16, 16)] = sbuf[r, pl.ds(k2 * 16, 16)] * dv

    pltpu.sync_copy(sbuf, agg_sh.at[pl.ds(s * SLC, SLC)])
    plsc.subcore_barrier()

    for g in range(prows_per_tile):
      pr = s * prows_per_tile + g
      for d, dref in enumerate((d1r_hbm, d2r_hbm)):
        pltpu.sync_copy(dref.at[pr], pidx)
        pltpu.make_async_copy(agg_sh.at[pidx.at[0]], prow_v, sem_g).start()
        pltpu.make_async_copy(agg_sh.at[pidx.at[0]], prow_v, sem_g).wait()
        pltpu.sync_copy(prow_v, pf_hbm.at[d, c, pl.ds(pr * 128, 128)])

  return k(srcp, dstp, gs_flat, dinv, d1r, d2r)


def _tc_dense(fp1, fp2, dti1, dti2, cell):
  RB = 64

  def body(fp1_ref, fp2_ref, dti1_ref, dti2_ref, cell_ref, out_ref):
    out_ref[:, 120:1144] = fp1_ref[...]
    out_ref[:, 1144:2168] = fp2_ref[...]
    out_ref[:, 2168:2680] = dti1_ref[...]
    out_ref[:, 2680:3192] = dti2_ref[...]
    out_ref[:, 3192:4146] = cell_ref[...]

  return pl.pallas_call(
      body,
      grid=(B // RB,),
      in_specs=[
          pl.BlockSpec((RB, 1024), lambda i: (i, 0)),
          pl.BlockSpec((RB, 1024), lambda i: (i, 0)),
          pl.BlockSpec((RB, 512), lambda i: (i, 0)),
          pl.BlockSpec((RB, 512), lambda i: (i, 0)),
          pl.BlockSpec((RB, 954), lambda i: (i, 0)),
      ],
      out_specs=pl.BlockSpec((RB, TOTAL), lambda i: (i, 0)),
      out_shape=jax.ShapeDtypeStruct((B, TOTAL), F32),
  )(fp1, fp2, dti1, dti2, cell)


def _tc_merge(pf, dense_out):
  RB = 256

  def body(pf_ref, old_ref, out_ref):
    v = pf_ref[...]
    d1 = jnp.concatenate([v[0, 0], v[0, 1]], axis=1)
    d2 = jnp.concatenate([v[1, 0], v[1, 1]], axis=1)
    out_ref[...] = jnp.concatenate(
        [d1[:, 0:60], d2[:, 0:60], old_ref[:, 120:128]], axis=1)

  return pl.pallas_call(
      body,
      grid=(B // RB,),
      in_specs=[
          pl.BlockSpec((2, NC, RB, 32), lambda i: (0, 0, i, 0)),
          pl.BlockSpec((RB, 128), lambda i: (i, 0)),
      ],
      out_specs=pl.BlockSpec((RB, 128), lambda i: (i, 0)),
      out_shape=jax.ShapeDtypeStruct((B, TOTAL), F32),
      input_output_aliases={1: 0},
  )(pf, dense_out)


def kernel(drug1_idx, drug2_idx, drug1_fp, drug2_fp, drug1_dti, drug2_dti,
           cell_feat, x, edge_index, W1, W2):
  src = edge_index[0].astype(jnp.int32)
  dst = edge_index[1].astype(jnp.int32)
  pad = jnp.full((EP - E,), DUMMY, jnp.int32)
  srcp = jnp.concatenate([src, pad]).reshape(ER, 128)
  dstp = jnp.concatenate([dst, pad]).reshape(ER, 128)
  x_pad = jnp.pad(x, ((0, NP - N), (0, 0)))
  w2p = jnp.pad(W2, ((0, 0), (0, OUTP - OUT)))
  d1r = jnp.ravel(drug1_idx).astype(jnp.int32).reshape(B // 128, 1, 128)
  d2r = jnp.ravel(drug2_idx).astype(jnp.int32).reshape(B // 128, 1, 128)

  dense_out = _tc_dense(drug1_fp, drug2_fp, drug1_dti, drug2_dti, cell_feat)
  degp = _sc_deg(dstp).reshape(NC, NP)
  dinv, xs = _tc_prep(x_pad, degp)
  xs_flat = xs.reshape(2 * NP, 64)
  agg1 = _sc_agg1(srcp, dstp, xs_flat)
  gs = _tc_mm(agg1, dinv, W1, w2p)
  gs_flat = gs.reshape(2 * NP, 32)
  pf = _sc_agg2(srcp, dstp, gs_flat, dinv.reshape(NP), d1r, d2r)
  return _tc_merge(pf, dense_out)


# speedup vs baseline: 1.8891x; 1.0518x over previous
import functools

import jax
import jax.numpy as jnp
from jax import lax
from jax.experimental import pallas as pl
from jax.experimental.pallas import tpu as pltpu
from jax.experimental.pallas import tpu_sc as plsc

N = 10000
NP = 10240
D = 128
H = 256
OUT = 60
OUTP = 64
B = 4096
E = 320000
EP = 327680
ER = EP // 128
DUMMY = 10208
NC = 2
NS = 16
SLC = NP // NS
K = 8
KP = 4
TOTAL = OUT + OUT + 1024 + 1024 + 512 + 512 + 954

F32 = jnp.float32


def _mesh():
  return plsc.VectorSubcoreMesh(core_axis_name="c", subcore_axis_name="s")


def _sc_deg(dstp):
  rows_per_tile = ER // (NC * NS)
  nsg = rows_per_tile // K

  @functools.partial(
      pl.kernel,
      mesh=_mesh(),
      compiler_params=pltpu.CompilerParams(use_tc_tiling_on_sc=False),
      out_type=jax.ShapeDtypeStruct((NC * NP,), F32),
      scratch_types=[
          pltpu.VMEM((K, 128), jnp.int32),
          pltpu.VMEM((128,), F32),
          pltpu.VMEM((SLC,), F32),
          pltpu.VMEM_SHARED((NP,), F32),
          pltpu.SemaphoreType.DMA,
      ],
  )
  def deg_kernel(dstp_hbm, out_hbm, idx_v, ones_v, zbuf_v, deg_sh, sem):
    c = lax.axis_index("c")
    s = lax.axis_index("s")

    @pl.loop(0, SLC // 16)
    def _(i):
      zbuf_v[pl.ds(i * 16, 16)] = jnp.zeros((16,), F32)

    @pl.loop(0, 128 // 16)
    def _(i):
      ones_v[pl.ds(i * 16, 16)] = jnp.ones((16,), F32)

    pltpu.sync_copy(zbuf_v, deg_sh.at[pl.ds(s * SLC, SLC)])
    plsc.subcore_barrier()

    base = (c * NS + s) * rows_per_tile

    @pl.loop(0, nsg)
    def _(g):
      pltpu.sync_copy(dstp_hbm.at[pl.ds(base + g * K, K)], idx_v)
      for j in range(K):
        pltpu.make_async_copy(ones_v, deg_sh.at[idx_v.at[j]], sem).start(
            add=True)
      for j in range(K):
        pltpu.make_async_copy(ones_v, deg_sh.at[idx_v.at[j]], sem).wait()

    plsc.subcore_barrier()
    pltpu.sync_copy(deg_sh.at[pl.ds(s * SLC, SLC)],
                    out_hbm.at[pl.ds(c * NP + s * SLC, SLC)])

  return deg_kernel(dstp)


def _tc_prep(x, degp):
  def body(deg_ref, x_ref, dinv_ref, xs_ref):
    deg = deg_ref[0, :] + deg_ref[1, :]
    dinv = lax.rsqrt(jnp.maximum(deg, 1.0))
    dinv_ref[...] = dinv[:, None]
    xs = x_ref[...] * dinv[:, None]
    xs_ref[...] = jnp.stack([xs[:, :64], xs[:, 64:]], axis=0)

  return pl.pallas_call(
      body,
      grid=(NS,),
      in_specs=[
          pl.BlockSpec((NC, SLC), lambda i: (0, i)),
          pl.BlockSpec((SLC, D), lambda i: (i, 0)),
      ],
      out_specs=[
          pl.BlockSpec((SLC, 1), lambda i: (i, 0)),
          pl.BlockSpec((NC, SLC, 64), lambda i: (0, i, 0)),
      ],
      out_shape=[
          jax.ShapeDtypeStruct((NP, 1), F32),
          jax.ShapeDtypeStruct((NC, NP, 64), F32),
      ],
  )(degp, x)


def _zero_slice(zbuf, agg_sh, s, w):
  @pl.loop(0, 64)
  def _(i):
    for k2 in range(w // 16):
      zbuf[i, pl.ds(k2 * 16, 16)] = jnp.zeros((16,), F32)

  for t in range(SLC // 64):
    pltpu.sync_copy(zbuf, agg_sh.at[pl.ds(s * SLC + t * 64, 64)])


def _edge_pipeline(c, s, src_hbm, dst_hbm, tab_sh, isrc, idst, rows0,
                   rows1, agg_sh, sem_g, sem_i, sem_s0, sem_s1):
  base = s * (ER // NS)
  IG = 16
  nsg = (ER // NS) // IG
  nw = IG // 2
  bufs = ((rows0, sem_s0), (rows1, sem_s1))

  def idx_copy(g, pg, start):
    r0 = jnp.minimum(base + g * IG, ER - IG)
    a = pltpu.make_async_copy(src_hbm.at[pl.ds(r0, IG)], isrc.at[pg], sem_i)
    b = pltpu.make_async_copy(dst_hbm.at[pl.ds(r0, IG)], idst.at[pg], sem_i)
    if start:
      a.start()
      b.start()
    else:
      a.wait()
      b.wait()

  @pl.loop(0, 128)
  def _(i):
    for j in range(2):
      for k2 in range(rows0.shape[2] // 16):
        rows0[j, i, pl.ds(k2 * 16, 16)] = jnp.zeros((16,), F32)
        rows1[j, i, pl.ds(k2 * 16, 16)] = jnp.zeros((16,), F32)

  for j in range(2):
    for kk in range(8):
      idst[1, j, pl.ds(kk * 16, 16)] = jnp.zeros((16,), jnp.int32)
  for p in range(2):
    rows_b, sem_sb = bufs[p]
    for j in range(2):
      pltpu.make_async_copy(rows_b.at[j], agg_sh.at[idst.at[1, j]],
                            sem_sb).start(add=True)

  idx_copy(0, 0, True)

  def do_sg(g, pg):
    qg = 1 - pg
    idx_copy(g, pg, False)
    for w in range(nw):
      rows_b, sem_sb = bufs[w % 2]
      for j in range(2):
        pltpu.make_async_copy(rows_b.at[j], agg_sh.at[idst.at[1, j]],
                              sem_sb).wait()
      if w == 1:
        idx_copy(g + 1, qg, True)
      for j in range(2):
        pltpu.make_async_copy(tab_sh.at[isrc.at[pg, 2 * w + j]],
                              rows_b.at[j], sem_g).start()
      for j in range(2):
        pltpu.make_async_copy(tab_sh.at[isrc.at[pg, 2 * w + j]],
                              rows_b.at[j], sem_g).wait()
        pltpu.make_async_copy(rows_b.at[j],
                              agg_sh.at[idst.at[pg, 2 * w + j]],
                              sem_sb).start(add=True)

  @pl.loop(0, nsg // 2)
  def _(t):
    do_sg(2 * t, 0)
    do_sg(2 * t + 1, 1)

  idx_copy(nsg, 0, False)
  for p in range(2):
    rows_b, sem_sb = bufs[p]
    for j in range(2):
      pltpu.make_async_copy(rows_b.at[j], agg_sh.at[idst.at[1, j]],
                            sem_sb).wait()


def _sc_agg1(srcp, dstp, xs_flat):
  W = 64

  @functools.partial(
      pl.kernel,
      mesh=_mesh(),
      compiler_params=pltpu.CompilerParams(use_tc_tiling_on_sc=False),
      out_type=jax.ShapeDtypeStruct((NC, NP, 64), F32),
      scratch_types=[
          pltpu.VMEM((2, 16, 128), jnp.int32),
          pltpu.VMEM((2, 16, 128), jnp.int32),
          pltpu.VMEM((2, 128, W), F32),
          pltpu.VMEM((2, 128, W), F32),
          pltpu.VMEM((64, W), F32),
          pltpu.VMEM_SHARED((NP, W), F32),
          pltpu.VMEM_SHARED((NP, W), F32),
          pltpu.SemaphoreType.DMA,
          pltpu.SemaphoreType.DMA,
          pltpu.SemaphoreType.DMA,
          pltpu.SemaphoreType.DMA,
      ],
  )
  def k(srcp_hbm, dstp_hbm, xs_hbm, out_hbm, isrc, idst, rows0, rows1,
        zbuf, tab_sh, agg_sh, sem_g, sem_i, sem_s0, sem_s1):
    c = lax.axis_index("c")
    s = lax.axis_index("s")

    _zero_slice(zbuf, agg_sh, s, W)
    pltpu.sync_copy(xs_hbm.at[pl.ds(c * NP + s * SLC, SLC)],
                    tab_sh.at[pl.ds(s * SLC, SLC)])
    plsc.subcore_barrier()
    _edge_pipeline(c, s, srcp_hbm, dstp_hbm, tab_sh, isrc, idst, rows0,
                   rows1, agg_sh, sem_g, sem_i, sem_s0, sem_s1)
    plsc.subcore_barrier()
    pltpu.sync_copy(agg_sh.at[pl.ds(s * SLC, SLC)],
                    out_hbm.at[c, pl.ds(s * SLC, SLC)])

  return k(srcp, dstp, xs_flat)


def _tc_mm(agg1, dinv, w1, w2p):
  def body(agg_ref, dinv_ref, w1_ref, w2_ref, gs_ref):
    dv = dinv_ref[...]
    av = agg_ref[...]
    a = jnp.concatenate([av[0], av[1]], axis=1) * dv
    h = jnp.dot(a, w1_ref[...], precision=lax.Precision.HIGHEST,
                preferred_element_type=F32)
    h = jnp.maximum(h, 0.0)
    g = jnp.dot(h, w2_ref[...], precision=lax.Precision.HIGHEST,
                preferred_element_type=F32)
    gs = g * dv
    gs_ref[...] = jnp.stack([gs[:, :32], gs[:, 32:]], axis=0)

  return pl.pallas_call(
      body,
      grid=(NS,),
      in_specs=[
          pl.BlockSpec((NC, SLC, 64), lambda i: (0, i, 0)),
          pl.BlockSpec((SLC, 1), lambda i: (i, 0)),
          pl.BlockSpec((D, H), lambda i: (0, 0)),
          pl.BlockSpec((H, OUTP), lambda i: (0, 0)),
      ],
      out_specs=pl.BlockSpec((NC, SLC, 32), lambda i: (0, i, 0)),
      out_shape=jax.ShapeDtypeStruct((NC, NP, 32), F32),
  )(agg1, dinv, w1, w2p)


def _sc_agg2(srcp, dstp, gs_flat, dinv, d1r, d2r):
  W = 32
  prows_per_tile = B // 128 // NS

  @functools.partial(
      pl.kernel,
      mesh=_mesh(),
      compiler_params=pltpu.CompilerParams(use_tc_tiling_on_sc=False),
      out_type=jax.ShapeDtypeStruct((2, NC, B, 32), F32),
      scratch_types=[
          pltpu.VMEM((2, 16, 128), jnp.int32),
          pltpu.VMEM((2, 16, 128), jnp.int32),
          pltpu.VMEM((2, 128, W), F32),
          pltpu.VMEM((2, 128, W), F32),
          pltpu.VMEM((64, W), F32),
          pltpu.VMEM((SLC, W), F32),
          pltpu.VMEM((SLC,), F32),
          pltpu.VMEM((1, 128), jnp.int32),
          pltpu.VMEM((128, W), F32),
          pltpu.VMEM_SHARED((NP, W), F32),
          pltpu.VMEM_SHARED((NP, W), F32),
          pltpu.SemaphoreType.DMA,
          pltpu.SemaphoreType.DMA,
          pltpu.SemaphoreType.DMA,
          pltpu.SemaphoreType.DMA,
      ],
  )
  def k(srcp_hbm, dstp_hbm, gs_hbm, dinv_hbm, d1r_hbm, d2r_hbm, pf_hbm,
        isrc, idst, rows0, rows1, zbuf, sbuf, dinv_v, pidx, prow_v,
        tab_sh, agg_sh, sem_g, sem_i, sem_s0, sem_s1):
    c = lax.axis_index("c")
    s = lax.axis_index("s")

    _zero_slice(zbuf, agg_sh, s, W)
    pltpu.sync_copy(gs_hbm.at[pl.ds(c * NP + s * SLC, SLC)],
                    tab_sh.at[pl.ds(s * SLC, SLC)])
    plsc.subcore_barrier()
    _edge_pipeline(c, s, srcp_hbm, dstp_hbm, tab_sh, isrc, idst, rows0,
                   rows1, agg_sh, sem_g, sem_i, sem_s0, sem_s1)
    plsc.subcore_barrier()

    pltpu.sync_copy(agg_sh.at[pl.ds(s * SLC, SLC)], sbuf)
    pltpu.sync_copy(dinv_hbm.at[pl.ds(s * SLC, SLC)], dinv_v)

    @pl.loop(0, SLC // 16)
    def _(q):
      dv16 = dinv_v[pl.ds(q * 16, 16)]
      for j in range(16):
        r = q * 16 + j
        dv = dv16[j]
        for k2 in range(W // 16):
          sbuf[r, pl.ds(k2 * 16, 16)] = sbuf[r, pl.ds(k2 * 16, 16)] * dv

    pltpu.sync_copy(sbuf, agg_sh.at[pl.ds(s * SLC, SLC)])
    plsc.subcore_barrier()

    for g in range(prows_per_tile):
      pr = s * prows_per_tile + g
      for d, dref in enumerate((d1r_hbm, d2r_hbm)):
        pltpu.sync_copy(dref.at[pr], pidx)
        pltpu.make_async_copy(agg_sh.at[pidx.at[0]], prow_v, sem_g).start()
        pltpu.make_async_copy(agg_sh.at[pidx.at[0]], prow_v, sem_g).wait()
        pltpu.sync_copy(prow_v, pf_hbm.at[d, c, pl.ds(pr * 128, 128)])

  return k(srcp, dstp, gs_flat, dinv, d1r, d2r)


def _tc_dense(fp1, fp2, dti1, dti2, cell):
  RB = 64

  def body(fp1_ref, fp2_ref, dti1_ref, dti2_ref, cell_ref, out_ref):
    out_ref[:, 120:1144] = fp1_ref[...]
    out_ref[:, 1144:2168] = fp2_ref[...]
    out_ref[:, 2168:2680] = dti1_ref[...]
    out_ref[:, 2680:3192] = dti2_ref[...]
    out_ref[:, 3192:4146] = cell_ref[...]

  return pl.pallas_call(
      body,
      grid=(B // RB,),
      in_specs=[
          pl.BlockSpec((RB, 1024), lambda i: (i, 0)),
          pl.BlockSpec((RB, 1024), lambda i: (i, 0)),
          pl.BlockSpec((RB, 512), lambda i: (i, 0)),
          pl.BlockSpec((RB, 512), lambda i: (i, 0)),
          pl.BlockSpec((RB, 954), lambda i: (i, 0)),
      ],
      out_specs=pl.BlockSpec((RB, TOTAL), lambda i: (i, 0)),
      out_shape=jax.ShapeDtypeStruct((B, TOTAL), F32),
  )(fp1, fp2, dti1, dti2, cell)


def _tc_merge(pf, dense_out):
  RB = 256

  def body(pf_ref, old_ref, out_ref):
    v = pf_ref[...]
    d1 = jnp.concatenate([v[0, 0], v[0, 1]], axis=1)
    d2 = jnp.concatenate([v[1, 0], v[1, 1]], axis=1)
    out_ref[...] = jnp.concatenate(
        [d1[:, 0:60], d2[:, 0:60], old_ref[:, 120:128]], axis=1)

  return pl.pallas_call(
      body,
      grid=(B // RB,),
      in_specs=[
          pl.BlockSpec((2, NC, RB, 32), lambda i: (0, 0, i, 0)),
          pl.BlockSpec((RB, 128), lambda i: (i, 0)),
      ],
      out_specs=pl.BlockSpec((RB, 128), lambda i: (i, 0)),
      out_shape=jax.ShapeDtypeStruct((B, TOTAL), F32),
      input_output_aliases={1: 0},
  )(pf, dense_out)


def kernel(drug1_idx, drug2_idx, drug1_fp, drug2_fp, drug1_dti, drug2_dti,
           cell_feat, x, edge_index, W1, W2):
  src = edge_index[0].astype(jnp.int32)
  dst = edge_index[1].astype(jnp.int32)
  pad = jnp.full((EP - E,), DUMMY, jnp.int32)
  srcp = jnp.concatenate([src, pad]).reshape(ER, 128)
  dstp = jnp.concatenate([dst, pad]).reshape(ER, 128)
  w2p = jnp.pad(W2, ((0, 0), (0, OUTP - OUT)))
  d1r = jnp.ravel(drug1_idx).astype(jnp.int32).reshape(B // 128, 1, 128)
  d2r = jnp.ravel(drug2_idx).astype(jnp.int32).reshape(B // 128, 1, 128)

  dense_out = _tc_dense(drug1_fp, drug2_fp, drug1_dti, drug2_dti, cell_feat)
  degp = _sc_deg(dstp).reshape(NC, NP)
  dinv, xs = _tc_prep(x, degp)
  xs_flat = xs.reshape(2 * NP, 64)
  agg1 = _sc_agg1(srcp, dstp, xs_flat)
  gs = _tc_mm(agg1, dinv, W1, w2p)
  gs_flat = gs.reshape(2 * NP, 32)
  pf = _sc_agg2(srcp, dstp, gs_flat, dinv.reshape(NP), d1r, d2r)
  return _tc_merge(pf, dense_out)


# speedup vs baseline: 2.2992x; 1.2171x over previous
import functools

import jax
import jax.numpy as jnp
from jax import lax
from jax.experimental import pallas as pl
from jax.experimental.pallas import tpu as pltpu
from jax.experimental.pallas import tpu_sc as plsc

N = 10000
NP = 10240
D = 128
H = 256
OUT = 60
OUTP = 64
B = 4096
E = 320000
EP = 327680
ER = EP // 128
DUMMY = 10208
NC = 2
NS = 16
SLC = NP // NS
K = 8
KP = 4
TOTAL = OUT + OUT + 1024 + 1024 + 512 + 512 + 954

F32 = jnp.float32


def _mesh():
  return plsc.VectorSubcoreMesh(core_axis_name="c", subcore_axis_name="s")


def _sc_deg(dstp):
  rows_per_tile = ER // (NC * NS)
  nsg = rows_per_tile // K

  @functools.partial(
      pl.kernel,
      mesh=_mesh(),
      compiler_params=pltpu.CompilerParams(use_tc_tiling_on_sc=False),
      out_type=jax.ShapeDtypeStruct((NC * NP,), F32),
      scratch_types=[
          pltpu.VMEM((K, 128), jnp.int32),
          pltpu.VMEM((128,), F32),
          pltpu.VMEM((SLC,), F32),
          pltpu.VMEM_SHARED((NP,), F32),
          pltpu.SemaphoreType.DMA,
      ],
  )
  def deg_kernel(dstp_hbm, out_hbm, idx_v, ones_v, zbuf_v, deg_sh, sem):
    c = lax.axis_index("c")
    s = lax.axis_index("s")

    @pl.loop(0, SLC // 16)
    def _(i):
      zbuf_v[pl.ds(i * 16, 16)] = jnp.zeros((16,), F32)

    @pl.loop(0, 128 // 16)
    def _(i):
      ones_v[pl.ds(i * 16, 16)] = jnp.ones((16,), F32)

    pltpu.sync_copy(zbuf_v, deg_sh.at[pl.ds(s * SLC, SLC)])
    plsc.subcore_barrier()

    base = (c * NS + s) * rows_per_tile

    @pl.loop(0, nsg)
    def _(g):
      pltpu.sync_copy(dstp_hbm.at[pl.ds(base + g * K, K)], idx_v)
      for j in range(K):
        pltpu.make_async_copy(ones_v, deg_sh.at[idx_v.at[j]], sem).start(
            add=True)
      for j in range(K):
        pltpu.make_async_copy(ones_v, deg_sh.at[idx_v.at[j]], sem).wait()

    plsc.subcore_barrier()
    pltpu.sync_copy(deg_sh.at[pl.ds(s * SLC, SLC)],
                    out_hbm.at[pl.ds(c * NP + s * SLC, SLC)])

  return deg_kernel(dstp)


def _tc_prep(x, degp):
  def body(deg_ref, x_ref, dinv_ref, xs_ref):
    deg = deg_ref[0, :] + deg_ref[1, :]
    dinv = lax.rsqrt(jnp.maximum(deg, 1.0))
    dinv_ref[...] = dinv[:, None]
    xs = x_ref[...] * dinv[:, None]
    xs_ref[...] = jnp.stack([xs[:, :64], xs[:, 64:]], axis=0)

  return pl.pallas_call(
      body,
      grid=(NS,),
      in_specs=[
          pl.BlockSpec((NC, SLC), lambda i: (0, i)),
          pl.BlockSpec((SLC, D), lambda i: (i, 0)),
      ],
      out_specs=[
          pl.BlockSpec((SLC, 1), lambda i: (i, 0)),
          pl.BlockSpec((NC, SLC, 64), lambda i: (0, i, 0)),
      ],
      out_shape=[
          jax.ShapeDtypeStruct((NP, 1), F32),
          jax.ShapeDtypeStruct((NC, NP, 64), F32),
      ],
  )(degp, x)


def _zero_slice(zbuf, agg_sh, s, w):
  @pl.loop(0, 64)
  def _(i):
    for k2 in range(w // 16):
      zbuf[i, pl.ds(k2 * 16, 16)] = jnp.zeros((16,), F32)

  for t in range(SLC // 64):
    pltpu.sync_copy(zbuf, agg_sh.at[pl.ds(s * SLC + t * 64, 64)])


def _edge_pipeline(c, s, src_hbm, dst_hbm, tab_sh, isrc, idst, rows0,
                   rows1, agg_sh, sem_g, sem_i, sem_s0, sem_s1):
  base = s * (ER // NS)
  IG = 16
  nsg = (ER // NS) // IG
  nw = IG // 2
  bufs = ((rows0, sem_s0), (rows1, sem_s1))

  def idx_copy(g, pg, start):
    r0 = jnp.minimum(base + g * IG, ER - IG)
    a = pltpu.make_async_copy(src_hbm.at[pl.ds(r0, IG)], isrc.at[pg], sem_i)
    b = pltpu.make_async_copy(dst_hbm.at[pl.ds(r0, IG)], idst.at[pg], sem_i)
    if start:
      a.start()
      b.start()
    else:
      a.wait()
      b.wait()

  @pl.loop(0, 128)
  def _(i):
    for j in range(2):
      for k2 in range(rows0.shape[2] // 16):
        rows0[j, i, pl.ds(k2 * 16, 16)] = jnp.zeros((16,), F32)
        rows1[j, i, pl.ds(k2 * 16, 16)] = jnp.zeros((16,), F32)

  for j in range(2):
    for kk in range(8):
      idst[1, j, pl.ds(kk * 16, 16)] = jnp.zeros((16,), jnp.int32)
  for p in range(2):
    rows_b, sem_sb = bufs[p]
    for j in range(2):
      pltpu.make_async_copy(rows_b.at[j], agg_sh.at[idst.at[1, j]],
                            sem_sb).start(add=True)

  idx_copy(0, 0, True)

  def do_sg(g, pg):
    qg = 1 - pg
    idx_copy(g, pg, False)
    for w in range(nw):
      rows_b, sem_sb = bufs[w % 2]
      for j in range(2):
        pltpu.make_async_copy(rows_b.at[j], agg_sh.at[idst.at[1, j]],
                              sem_sb).wait()
      if w == 1:
        idx_copy(g + 1, qg, True)
      for j in range(2):
        pltpu.make_async_copy(tab_sh.at[isrc.at[pg, 2 * w + j]],
                              rows_b.at[j], sem_g).start()
      for j in range(2):
        pltpu.make_async_copy(tab_sh.at[isrc.at[pg, 2 * w + j]],
                              rows_b.at[j], sem_g).wait()
        pltpu.make_async_copy(rows_b.at[j],
                              agg_sh.at[idst.at[pg, 2 * w + j]],
                              sem_sb).start(add=True)

  @pl.loop(0, nsg // 2)
  def _(t):
    do_sg(2 * t, 0)
    do_sg(2 * t + 1, 1)

  idx_copy(nsg, 0, False)
  for p in range(2):
    rows_b, sem_sb = bufs[p]
    for j in range(2):
      pltpu.make_async_copy(rows_b.at[j], agg_sh.at[idst.at[1, j]],
                            sem_sb).wait()


def _sc_agg1(srcp, dstp, xs_flat):
  W = 64

  @functools.partial(
      pl.kernel,
      mesh=_mesh(),
      compiler_params=pltpu.CompilerParams(use_tc_tiling_on_sc=False),
      out_type=jax.ShapeDtypeStruct((NC, NP, 64), F32),
      scratch_types=[
          pltpu.VMEM((2, 16, 128), jnp.int32),
          pltpu.VMEM((2, 16, 128), jnp.int32),
          pltpu.VMEM((2, 128, W), F32),
          pltpu.VMEM((2, 128, W), F32),
          pltpu.VMEM((64, W), F32),
          pltpu.VMEM_SHARED((NP, W), F32),
          pltpu.VMEM_SHARED((NP, W), F32),
          pltpu.SemaphoreType.DMA,
          pltpu.SemaphoreType.DMA,
          pltpu.SemaphoreType.DMA,
          pltpu.SemaphoreType.DMA,
      ],
  )
  def k(srcp_hbm, dstp_hbm, xs_hbm, out_hbm, isrc, idst, rows0, rows1,
        zbuf, tab_sh, agg_sh, sem_g, sem_i, sem_s0, sem_s1):
    c = lax.axis_index("c")
    s = lax.axis_index("s")

    _zero_slice(zbuf, agg_sh, s, W)
    pltpu.sync_copy(xs_hbm.at[pl.ds(c * NP + s * SLC, SLC)],
                    tab_sh.at[pl.ds(s * SLC, SLC)])
    plsc.subcore_barrier()
    _edge_pipeline(c, s, srcp_hbm, dstp_hbm, tab_sh, isrc, idst, rows0,
                   rows1, agg_sh, sem_g, sem_i, sem_s0, sem_s1)
    plsc.subcore_barrier()
    pltpu.sync_copy(agg_sh.at[pl.ds(s * SLC, SLC)],
                    out_hbm.at[c, pl.ds(s * SLC, SLC)])

  return k(srcp, dstp, xs_flat)


def _tc_mm(agg1, dinv, w1, w2p):
  def body(agg_ref, dinv_ref, w1_ref, w2_ref, gs_ref):
    dv = dinv_ref[...]
    av = agg_ref[...]
    a = jnp.concatenate([av[0], av[1]], axis=1) * dv
    h = jnp.dot(a, w1_ref[...], preferred_element_type=F32)
    h = jnp.maximum(h, 0.0)
    g = jnp.dot(h, w2_ref[...], preferred_element_type=F32)
    gs = g * dv
    gs_ref[...] = jnp.stack([gs[:, :32], gs[:, 32:]], axis=0)

  return pl.pallas_call(
      body,
      grid=(NS,),
      in_specs=[
          pl.BlockSpec((NC, SLC, 64), lambda i: (0, i, 0)),
          pl.BlockSpec((SLC, 1), lambda i: (i, 0)),
          pl.BlockSpec((D, H), lambda i: (0, 0)),
          pl.BlockSpec((H, OUTP), lambda i: (0, 0)),
      ],
      out_specs=pl.BlockSpec((NC, SLC, 32), lambda i: (0, i, 0)),
      out_shape=jax.ShapeDtypeStruct((NC, NP, 32), F32),
  )(agg1, dinv, w1, w2p)


def _sc_agg2(srcp, dstp, gs_flat, dinv, d1r, d2r):
  W = 32
  prows_per_tile = B // 128 // NS

  @functools.partial(
      pl.kernel,
      mesh=_mesh(),
      compiler_params=pltpu.CompilerParams(use_tc_tiling_on_sc=False),
      out_type=jax.ShapeDtypeStruct((2, NC, B, 32), F32),
      scratch_types=[
          pltpu.VMEM((2, 16, 128), jnp.int32),
          pltpu.VMEM((2, 16, 128), jnp.int32),
          pltpu.VMEM((2, 128, W), F32),
          pltpu.VMEM((2, 128, W), F32),
          pltpu.VMEM((64, W), F32),
          pltpu.VMEM((SLC, W), F32),
          pltpu.VMEM((SLC,), F32),
          pltpu.VMEM((1, 128), jnp.int32),
          pltpu.VMEM((128, W), F32),
          pltpu.VMEM_SHARED((NP, W), F32),
          pltpu.VMEM_SHARED((NP, W), F32),
          pltpu.SemaphoreType.DMA,
          pltpu.SemaphoreType.DMA,
          pltpu.SemaphoreType.DMA,
          pltpu.SemaphoreType.DMA,
      ],
  )
  def k(srcp_hbm, dstp_hbm, gs_hbm, dinv_hbm, d1r_hbm, d2r_hbm, pf_hbm,
        isrc, idst, rows0, rows1, zbuf, sbuf, dinv_v, pidx, prow_v,
        tab_sh, agg_sh, sem_g, sem_i, sem_s0, sem_s1):
    c = lax.axis_index("c")
    s = lax.axis_index("s")

    _zero_slice(zbuf, agg_sh, s, W)
    pltpu.sync_copy(gs_hbm.at[pl.ds(c * NP + s * SLC, SLC)],
                    tab_sh.at[pl.ds(s * SLC, SLC)])
    plsc.subcore_barrier()
    _edge_pipeline(c, s, srcp_hbm, dstp_hbm, tab_sh, isrc, idst, rows0,
                   rows1, agg_sh, sem_g, sem_i, sem_s0, sem_s1)
    plsc.subcore_barrier()

    pltpu.sync_copy(agg_sh.at[pl.ds(s * SLC, SLC)], sbuf)
    pltpu.sync_copy(dinv_hbm.at[pl.ds(s * SLC, SLC)], dinv_v)

    @pl.loop(0, SLC // 16)
    def _(q):
      dv16 = dinv_v[pl.ds(q * 16, 16)]
      for j in range(16):
        r = q * 16 + j
        dv = dv16[j]
        for k2 in range(W // 16):
          sbuf[r, pl.ds(k2 * 16, 16)] = sbuf[r, pl.ds(k2 * 16, 16)] * dv

    pltpu.sync_copy(sbuf, agg_sh.at[pl.ds(s * SLC, SLC)])
    plsc.subcore_barrier()

    for g in range(prows_per_tile):
      pr = s * prows_per_tile + g
      for d, dref in enumerate((d1r_hbm, d2r_hbm)):
        pltpu.sync_copy(dref.at[pr], pidx)
        pltpu.make_async_copy(agg_sh.at[pidx.at[0]], prow_v, sem_g).start()
        pltpu.make_async_copy(agg_sh.at[pidx.at[0]], prow_v, sem_g).wait()
        pltpu.sync_copy(prow_v, pf_hbm.at[d, c, pl.ds(pr * 128, 128)])

  return k(srcp, dstp, gs_flat, dinv, d1r, d2r)


def _tc_dense(fp1, fp2, dti1, dti2, cell_t):
  RB = 256

  def body(fp1_ref, fp2_ref, dti1_ref, dti2_ref, cell_ref, out_ref):
    out_ref[120:1144, :] = fp1_ref[...].T
    out_ref[1144:2168, :] = fp2_ref[...].T
    out_ref[2168:2680, :] = dti1_ref[...].T
    out_ref[2680:3192, :] = dti2_ref[...].T
    out_ref[3192:4146, :] = cell_ref[...]

  return pl.pallas_call(
      body,
      grid=(B // RB,),
      in_specs=[
          pl.BlockSpec((RB, 1024), lambda i: (i, 0)),
          pl.BlockSpec((RB, 1024), lambda i: (i, 0)),
          pl.BlockSpec((RB, 512), lambda i: (i, 0)),
          pl.BlockSpec((RB, 512), lambda i: (i, 0)),
          pl.BlockSpec((954, RB), lambda i: (0, i)),
      ],
      out_specs=pl.BlockSpec((TOTAL, RB), lambda i: (0, i)),
      out_shape=jax.ShapeDtypeStruct((TOTAL, B), F32),
  )(fp1, fp2, dti1, dti2, cell_t)


def _tc_merge(pf, dense_out):
  RB = 256

  def body(pf_ref, old_ref, out_ref):
    v = pf_ref[...]
    d1 = jnp.concatenate([v[0, 0], v[0, 1]], axis=1).T
    d2 = jnp.concatenate([v[1, 0], v[1, 1]], axis=1).T
    out_ref[...] = jnp.concatenate(
        [d1[0:60], d2[0:60], old_ref[120:128, :]], axis=0)

  return pl.pallas_call(
      body,
      grid=(B // RB,),
      in_specs=[
          pl.BlockSpec((2, NC, RB, 32), lambda i: (0, 0, i, 0)),
          pl.BlockSpec((128, RB), lambda i: (0, i)),
      ],
      out_specs=pl.BlockSpec((128, RB), lambda i: (0, i)),
      out_shape=jax.ShapeDtypeStruct((TOTAL, B), F32),
      input_output_aliases={1: 0},
  )(pf, dense_out)


def kernel(drug1_idx, drug2_idx, drug1_fp, drug2_fp, drug1_dti, drug2_dti,
           cell_feat, x, edge_index, W1, W2):
  src = edge_index[0].astype(jnp.int32)
  dst = edge_index[1].astype(jnp.int32)
  pad = jnp.full((EP - E,), DUMMY, jnp.int32)
  srcp = jnp.concatenate([src, pad]).reshape(ER, 128)
  dstp = jnp.concatenate([dst, pad]).reshape(ER, 128)
  w2p = jnp.pad(W2, ((0, 0), (0, OUTP - OUT)))
  d1r = jnp.ravel(drug1_idx).astype(jnp.int32).reshape(B // 128, 1, 128)
  d2r = jnp.ravel(drug2_idx).astype(jnp.int32).reshape(B // 128, 1, 128)

  dense_out = _tc_dense(drug1_fp, drug2_fp, drug1_dti, drug2_dti,
                        jnp.transpose(cell_feat))
  degp = _sc_deg(dstp).reshape(NC, NP)
  dinv, xs = _tc_prep(x, degp)
  xs_flat = xs.reshape(2 * NP, 64)
  agg1 = _sc_agg1(srcp, dstp, xs_flat)
  gs = _tc_mm(agg1, dinv, W1, w2p)
  gs_flat = gs.reshape(2 * NP, 32)
  pf = _sc_agg2(srcp, dstp, gs_flat, dinv.reshape(NP), d1r, d2r)
  return jnp.transpose(_tc_merge(pf, dense_out))


# speedup vs baseline: 2.3000x; 1.0003x over previous
import functools

import jax
import jax.numpy as jnp
from jax import lax
from jax.experimental import pallas as pl
from jax.experimental.pallas import tpu as pltpu
from jax.experimental.pallas import tpu_sc as plsc

N = 10000
NP = 10240
D = 128
H = 256
OUT = 60
OUTP = 64
B = 4096
E = 320000
EP = 327680
ER = EP // 128
DUMMY = 10208
NC = 2
NS = 16
SLC = NP // NS
K = 8
TOTAL = OUT + OUT + 1024 + 1024 + 512 + 512 + 954

F32 = jnp.float32


def _mesh():
  return plsc.VectorSubcoreMesh(core_axis_name="c", subcore_axis_name="s")


def _sc_deg(dstp):
  rows_per_tile = ER // (NC * NS)
  nsg = rows_per_tile // K

  @functools.partial(
      pl.kernel,
      mesh=_mesh(),
      compiler_params=pltpu.CompilerParams(use_tc_tiling_on_sc=False),
      out_type=jax.ShapeDtypeStruct((NC * NP,), F32),
      scratch_types=[
          pltpu.VMEM((K, 128), jnp.int32),
          pltpu.VMEM((128,), F32),
          pltpu.VMEM((SLC,), F32),
          pltpu.VMEM_SHARED((NP,), F32),
          pltpu.SemaphoreType.DMA,
      ],
  )
  def deg_kernel(dstp_hbm, out_hbm, idx_v, ones_v, zbuf_v, deg_sh, sem):
    c = lax.axis_index("c")
    s = lax.axis_index("s")

    @pl.loop(0, SLC // 16)
    def _(i):
      zbuf_v[pl.ds(i * 16, 16)] = jnp.zeros((16,), F32)

    @pl.loop(0, 128 // 16)
    def _(i):
      ones_v[pl.ds(i * 16, 16)] = jnp.ones((16,), F32)

    pltpu.sync_copy(zbuf_v, deg_sh.at[pl.ds(s * SLC, SLC)])
    plsc.subcore_barrier()

    base = (c * NS + s) * rows_per_tile

    @pl.loop(0, nsg)
    def _(g):
      pltpu.sync_copy(dstp_hbm.at[pl.ds(base + g * K, K)], idx_v)
      for j in range(K):
        pltpu.make_async_copy(ones_v, deg_sh.at[idx_v.at[j]], sem).start(
            add=True)
      for j in range(K):
        pltpu.make_async_copy(ones_v, deg_sh.at[idx_v.at[j]], sem).wait()

    plsc.subcore_barrier()
    pltpu.sync_copy(deg_sh.at[pl.ds(s * SLC, SLC)],
                    out_hbm.at[pl.ds(c * NP + s * SLC, SLC)])

  return deg_kernel(dstp)


def _tc_prep(x, degp):
  def body(deg_ref, x_ref, dinv_ref, xs_ref):
    deg = deg_ref[0, :] + deg_ref[1, :]
    dinv = lax.rsqrt(jnp.maximum(deg, 1.0))
    dinv_ref[...] = dinv[:, None]
    xs = x_ref[...] * dinv[:, None]
    xs_ref[...] = jnp.stack([xs[:, :64], xs[:, 64:]], axis=0)

  return pl.pallas_call(
      body,
      grid=(NS,),
      in_specs=[
          pl.BlockSpec((NC, SLC), lambda i: (0, i)),
          pl.BlockSpec((SLC, D), lambda i: (i, 0)),
      ],
      out_specs=[
          pl.BlockSpec((SLC, 1), lambda i: (i, 0)),
          pl.BlockSpec((NC, SLC, 64), lambda i: (0, i, 0)),
      ],
      out_shape=[
          jax.ShapeDtypeStruct((NP, 1), F32),
          jax.ShapeDtypeStruct((NC, NP, 64), F32),
      ],
  )(degp, x)


def _zero_slice(zbuf, agg_sh, s, w):
  @pl.loop(0, 64)
  def _(i):
    for k2 in range(w // 16):
      zbuf[i, pl.ds(k2 * 16, 16)] = jnp.zeros((16,), F32)

  for t in range(SLC // 64):
    pltpu.sync_copy(zbuf, agg_sh.at[pl.ds(s * SLC + t * 64, 64)])


def _edge_pipeline(c, s, src_hbm, dst_hbm, tab_sh, isrc, idst, rows0,
                   rows1, agg_sh, sem_g, sem_i, sem_s0, sem_s1):
  base = s * (ER // NS)
  IG = 16
  nsg = (ER // NS) // IG
  nw = IG // 2
  bufs = ((rows0, sem_s0), (rows1, sem_s1))

  def idx_copy(g, pg, start):
    r0 = jnp.minimum(base + g * IG, ER - IG)
    a = pltpu.make_async_copy(src_hbm.at[pl.ds(r0, IG)], isrc.at[pg], sem_i)
    b = pltpu.make_async_copy(dst_hbm.at[pl.ds(r0, IG)], idst.at[pg], sem_i)
    if start:
      a.start()
      b.start()
    else:
      a.wait()
      b.wait()

  @pl.loop(0, 128)
  def _(i):
    for j in range(2):
      for k2 in range(rows0.shape[2] // 16):
        rows0[j, i, pl.ds(k2 * 16, 16)] = jnp.zeros((16,), F32)
        rows1[j, i, pl.ds(k2 * 16, 16)] = jnp.zeros((16,), F32)

  for j in range(2):
    for kk in range(8):
      idst[1, j, pl.ds(kk * 16, 16)] = jnp.zeros((16,), jnp.int32)
  for p in range(2):
    rows_b, sem_sb = bufs[p]
    for j in range(2):
      pltpu.make_async_copy(rows_b.at[j], agg_sh.at[idst.at[1, j]],
                            sem_sb).start(add=True)

  idx_copy(0, 0, True)

  def do_sg(g, pg):
    qg = 1 - pg
    idx_copy(g, pg, False)
    for w in range(nw):
      rows_b, sem_sb = bufs[w % 2]
      for j in range(2):
        pltpu.make_async_copy(rows_b.at[j], agg_sh.at[idst.at[1, j]],
                              sem_sb).wait()
      if w == 1:
        idx_copy(g + 1, qg, True)
      for j in range(2):
        pltpu.make_async_copy(tab_sh.at[isrc.at[pg, 2 * w + j]],
                              rows_b.at[j], sem_g).start()
      for j in range(2):
        pltpu.make_async_copy(tab_sh.at[isrc.at[pg, 2 * w + j]],
                              rows_b.at[j], sem_g).wait()
        pltpu.make_async_copy(rows_b.at[j],
                              agg_sh.at[idst.at[pg, 2 * w + j]],
                              sem_sb).start(add=True)

  @pl.loop(0, nsg // 2)
  def _(t):
    do_sg(2 * t, 0)
    do_sg(2 * t + 1, 1)

  idx_copy(nsg, 0, False)
  for p in range(2):
    rows_b, sem_sb = bufs[p]
    for j in range(2):
      pltpu.make_async_copy(rows_b.at[j], agg_sh.at[idst.at[1, j]],
                            sem_sb).wait()


def _sc_agg1(srcp, dstp, xs_flat):
  W = 64

  @functools.partial(
      pl.kernel,
      mesh=_mesh(),
      compiler_params=pltpu.CompilerParams(use_tc_tiling_on_sc=False),
      out_type=jax.ShapeDtypeStruct((NC, NP, 64), F32),
      scratch_types=[
          pltpu.VMEM((2, 16, 128), jnp.int32),
          pltpu.VMEM((2, 16, 128), jnp.int32),
          pltpu.VMEM((2, 128, W), F32),
          pltpu.VMEM((2, 128, W), F32),
          pltpu.VMEM((64, W), F32),
          pltpu.VMEM_SHARED((NP, W), F32),
          pltpu.VMEM_SHARED((NP, W), F32),
          pltpu.SemaphoreType.DMA,
          pltpu.SemaphoreType.DMA,
          pltpu.SemaphoreType.DMA,
          pltpu.SemaphoreType.DMA,
      ],
  )
  def k(srcp_hbm, dstp_hbm, xs_hbm, out_hbm, isrc, idst, rows0, rows1,
        zbuf, tab_sh, agg_sh, sem_g, sem_i, sem_s0, sem_s1):
    c = lax.axis_index("c")
    s = lax.axis_index("s")

    _zero_slice(zbuf, agg_sh, s, W)
    pltpu.sync_copy(xs_hbm.at[pl.ds(c * NP + s * SLC, SLC)],
                    tab_sh.at[pl.ds(s * SLC, SLC)])
    plsc.subcore_barrier()
    _edge_pipeline(c, s, srcp_hbm, dstp_hbm, tab_sh, isrc, idst, rows0,
                   rows1, agg_sh, sem_g, sem_i, sem_s0, sem_s1)
    plsc.subcore_barrier()
    pltpu.sync_copy(agg_sh.at[pl.ds(s * SLC, SLC)],
                    out_hbm.at[c, pl.ds(s * SLC, SLC)])

  return k(srcp, dstp, xs_flat)


def _tc_mm(agg1, dinv, w1, w2p):
  def body(agg_ref, dinv_ref, w1_ref, w2_ref, gs_ref):
    dv = dinv_ref[...]
    av = agg_ref[...]
    a = jnp.concatenate([av[0], av[1]], axis=1) * dv
    h = jnp.dot(a, w1_ref[...], preferred_element_type=F32)
    h = jnp.maximum(h, 0.0)
    g = jnp.dot(h, w2_ref[...], preferred_element_type=F32)
    gs = g * dv
    gs_ref[...] = jnp.stack([gs[:, :32], gs[:, 32:]], axis=0)

  return pl.pallas_call(
      body,
      grid=(NS,),
      in_specs=[
          pl.BlockSpec((NC, SLC, 64), lambda i: (0, i, 0)),
          pl.BlockSpec((SLC, 1), lambda i: (i, 0)),
          pl.BlockSpec((D, H), lambda i: (0, 0)),
          pl.BlockSpec((H, OUTP), lambda i: (0, 0)),
      ],
      out_specs=pl.BlockSpec((NC, SLC, 32), lambda i: (0, i, 0)),
      out_shape=jax.ShapeDtypeStruct((NC, NP, 32), F32),
  )(agg1, dinv, w1, w2p)


def _sc_agg2(srcp, dstp, gs_flat, dinv, d1r, d2r):
  W = 32
  prows_per_tile = B // 128 // NS

  @functools.partial(
      pl.kernel,
      mesh=_mesh(),
      compiler_params=pltpu.CompilerParams(use_tc_tiling_on_sc=False),
      out_type=jax.ShapeDtypeStruct((2, NC, B, 32), F32),
      scratch_types=[
          pltpu.VMEM((2, 16, 128), jnp.int32),
          pltpu.VMEM((2, 16, 128), jnp.int32),
          pltpu.VMEM((2, 128, W), F32),
          pltpu.VMEM((2, 128, W), F32),
          pltpu.VMEM((64, W), F32),
          pltpu.VMEM((SLC, W), F32),
          pltpu.VMEM((SLC,), F32),
          pltpu.VMEM((1, 128), jnp.int32),
          pltpu.VMEM((128, W), F32),
          pltpu.VMEM_SHARED((NP, W), F32),
          pltpu.VMEM_SHARED((NP, W), F32),
          pltpu.SemaphoreType.DMA,
          pltpu.SemaphoreType.DMA,
          pltpu.SemaphoreType.DMA,
          pltpu.SemaphoreType.DMA,
      ],
  )
  def k(srcp_hbm, dstp_hbm, gs_hbm, dinv_hbm, d1r_hbm, d2r_hbm, pf_hbm,
        isrc, idst, rows0, rows1, zbuf, sbuf, dinv_v, pidx, prow_v,
        tab_sh, agg_sh, sem_g, sem_i, sem_s0, sem_s1):
    c = lax.axis_index("c")
    s = lax.axis_index("s")

    _zero_slice(zbuf, agg_sh, s, W)
    pltpu.sync_copy(gs_hbm.at[pl.ds(c * NP + s * SLC, SLC)],
                    tab_sh.at[pl.ds(s * SLC, SLC)])
    plsc.subcore_barrier()
    _edge_pipeline(c, s, srcp_hbm, dstp_hbm, tab_sh, isrc, idst, rows0,
                   rows1, agg_sh, sem_g, sem_i, sem_s0, sem_s1)
    plsc.subcore_barrier()

    pltpu.sync_copy(agg_sh.at[pl.ds(s * SLC, SLC)], sbuf)
    pltpu.sync_copy(dinv_hbm.at[pl.ds(s * SLC, SLC)], dinv_v)

    @pl.loop(0, SLC // 16)
    def _(q):
      dv16 = dinv_v[pl.ds(q * 16, 16)]
      for j in range(16):
        r = q * 16 + j
        dv = dv16[j]
        for k2 in range(W // 16):
          sbuf[r, pl.ds(k2 * 16, 16)] = sbuf[r, pl.ds(k2 * 16, 16)] * dv

    pltpu.sync_copy(sbuf, agg_sh.at[pl.ds(s * SLC, SLC)])
    plsc.subcore_barrier()

    for g in range(prows_per_tile):
      pr = s * prows_per_tile + g
      for d, dref in enumerate((d1r_hbm, d2r_hbm)):
        pltpu.sync_copy(dref.at[pr], pidx)
        pltpu.make_async_copy(agg_sh.at[pidx.at[0]], prow_v, sem_g).start()
        pltpu.make_async_copy(agg_sh.at[pidx.at[0]], prow_v, sem_g).wait()
        pltpu.sync_copy(prow_v, pf_hbm.at[d, c, pl.ds(pr * 128, 128)])

  return k(srcp, dstp, gs_flat, dinv, d1r, d2r)


def _tc_dense(fp1, fp2, dti1, dti2, cell_t):
  RB = 256

  def body(fp1_ref, fp2_ref, dti1_ref, dti2_ref, cell_ref, out_ref):
    out_ref[120:1144, :] = fp1_ref[...].T
    out_ref[1144:2168, :] = fp2_ref[...].T
    out_ref[2168:2680, :] = dti1_ref[...].T
    out_ref[2680:3192, :] = dti2_ref[...].T
    out_ref[3192:4146, :] = cell_ref[...]

  return pl.pallas_call(
      body,
      grid=(B // RB,),
      in_specs=[
          pl.BlockSpec((RB, 1024), lambda i: (i, 0)),
          pl.BlockSpec((RB, 1024), lambda i: (i, 0)),
          pl.BlockSpec((RB, 512), lambda i: (i, 0)),
          pl.BlockSpec((RB, 512), lambda i: (i, 0)),
          pl.BlockSpec((954, RB), lambda i: (0, i)),
      ],
      out_specs=pl.BlockSpec((TOTAL, RB), lambda i: (0, i)),
      out_shape=jax.ShapeDtypeStruct((TOTAL, B), F32),
  )(fp1, fp2, dti1, dti2, cell_t)


def _tc_merge(pf, dense_out):
  RB = 256

  def body(pf_ref, old_ref, out_ref):
    v = pf_ref[...]
    d1 = jnp.concatenate([v[0, 0], v[0, 1]], axis=1).T
    d2 = jnp.concatenate([v[1, 0], v[1, 1]], axis=1).T
    out_ref[...] = jnp.concatenate(
        [d1[0:60], d2[0:60], old_ref[120:128, :]], axis=0)

  return pl.pallas_call(
      body,
      grid=(B // RB,),
      in_specs=[
          pl.BlockSpec((2, NC, RB, 32), lambda i: (0, 0, i, 0)),
          pl.BlockSpec((128, RB), lambda i: (0, i)),
      ],
      out_specs=pl.BlockSpec((128, RB), lambda i: (0, i)),
      out_shape=jax.ShapeDtypeStruct((TOTAL, B), F32),
      input_output_aliases={1: 0},
  )(pf, dense_out)


def kernel(drug1_idx, drug2_idx, drug1_fp, drug2_fp, drug1_dti, drug2_dti,
           cell_feat, x, edge_index, W1, W2):
  src = edge_index[0].astype(jnp.int32)
  dst = edge_index[1].astype(jnp.int32)
  pad = jnp.full((EP - E,), DUMMY, jnp.int32)
  srcp = jnp.concatenate([src, pad]).reshape(ER, 128)
  dstp = jnp.concatenate([dst, pad]).reshape(ER, 128)
  w2p = jnp.pad(W2, ((0, 0), (0, OUTP - OUT)))
  d1r = jnp.ravel(drug1_idx).astype(jnp.int32).reshape(B // 128, 1, 128)
  d2r = jnp.ravel(drug2_idx).astype(jnp.int32).reshape(B // 128, 1, 128)

  dense_out = _tc_dense(drug1_fp, drug2_fp, drug1_dti, drug2_dti,
                        jnp.transpose(cell_feat))
  degp = _sc_deg(dstp).reshape(NC, NP)
  dinv, xs = _tc_prep(x, degp)
  xs_flat = xs.reshape(2 * NP, 64)
  agg1 = _sc_agg1(srcp, dstp, xs_flat)
  gs = _tc_mm(agg1, dinv, W1, w2p)
  gs_flat = gs.reshape(2 * NP, 32)
  pf = _sc_agg2(srcp, dstp, gs_flat, dinv.reshape(NP), d1r, d2r)
  return jnp.transpose(_tc_merge(pf, dense_out))
